# baseline (speedup 1.0000x reference)
import jax
import jax.numpy as jnp
from jax import lax
from jax.experimental import pallas as pl
from jax.experimental.pallas import tpu as pltpu, tpu_sc as plsc

N_NODE = 10000
E = 320000
D = 128
R_PAD = 10240
B_PAD = 10240
QCHUNK = 64
CHUNK = 32
NUM_CHUNKS = E // CHUNK
NW = 32
T_ITER = 314
AGG = 10000
ROWS_PER_TILE = 632

PERM = [32 * k + (2 * j if j < 16 else 2 * (j - 16) + 1)
        for k in range(4) for j in range(32)]


def _sub_body(x_ref, w_ref, o_ref):
    x = x_ref[...]
    o_ref[:, :D] = jnp.dot(x, w_ref[...],
                           preferred_element_type=jnp.float32
                           ).astype(jnp.bfloat16)
    o_ref[:, D:] = x.astype(jnp.bfloat16)


def _rel_body(x_ref, wr_ref, wqr_ref, bqr_ref, or_ref, oc_ref):
    x = x_ref[...]
    or_ref[:, :D] = jnp.dot(x, wr_ref[...],
                            preferred_element_type=jnp.float32
                            ).astype(jnp.bfloat16)
    or_ref[:, D:] = x.astype(jnp.bfloat16)
    oc_ref[...] = (jnp.dot(x, wqr_ref[...], preferred_element_type=jnp.float32)
                   + bqr_ref[...])


def _fin_body(p_ref, w_ref, o_ref):
    x = p_ref[0] + p_ref[1]
    o_ref[...] = jnp.dot(x, w_ref[...], preferred_element_type=jnp.float32)


def _qgather_body(tabc_hbm, qrel_hbm, out_hbm, qidx_v, rows_v, sem):
    c = lax.axis_index("c")
    s = lax.axis_index("s")
    wid = s * 2 + c
    base = wid * (B_PAD // NW)

    def blk(j, carry):
        off = base + j * QCHUNK
        pltpu.sync_copy(qrel_hbm.at[pl.ds(off, QCHUNK)], qidx_v)
        pltpu.async_copy(tabc_hbm.at[qidx_v], rows_v, sem).wait()
        pltpu.sync_copy(rows_v, out_hbm.at[pl.ds(off, QCHUNK)])
        return carry

    lax.fori_loop(0, (B_PAD // NW) // QCHUNK, blk, 0)


def _sc_body(edges_hbm, subt_hbm, relt_hbm, tabC2_hbm, wvec_hbm, out_hbm,
             edg0, edg1, cols0, cols1, obj0, obj1,
             sr0, sr1, c0, c1, m_v, w_v, agg_sh,
             sem_e0, sem_e1, sem_g0, sem_g1):
    c = lax.axis_index("c")
    s = lax.axis_index("s")
    wid = s * 2 + c

    edg = (edg0, edg1)
    cols = (cols0, cols1)
    obj = (obj0, obj1)
    sr = (sr0, sr1)
    cb = (c0, c1)
    sem_e = (sem_e0, sem_e1)
    sem_g = (sem_g0, sem_g1)

    pltpu.sync_copy(wvec_hbm, w_v)
    row0 = s * ROWS_PER_TILE

    zf = jnp.zeros((16,), jnp.float32)

    def zero_body(e, cc):
        for k in range(8):
            m_v[e, pl.ds(16 * k, 16)] = zf
        return cc

    lax.fori_loop(0, CHUNK, zero_body, 0)

    @pl.when(s < 15)
    def _():
        for t in range(19):
            pltpu.sync_copy(m_v, agg_sh.at[pl.ds(row0 + 32 * t, 32)])
        pltpu.sync_copy(m_v.at[pl.ds(0, 24)],
                        agg_sh.at[pl.ds(row0 + 608, 24)])

    @pl.when(s == 15)
    def _():
        for t in range(16):
            pltpu.sync_copy(m_v, agg_sh.at[pl.ds(row0 + 32 * t, 32)])
        pltpu.sync_copy(m_v.at[pl.ds(0, 8)],
                        agg_sh.at[pl.ds(row0 + 512, 8)])

    plsc.subcore_barrier()

    wv = [w_v[pl.ds(16 * k, 16)] for k in range(8)]
    sbv = w_v[pl.ds(128, 16)]

    def fire_edges(ii, b):
        chunk = wid + ii * NW
        pltpu.async_copy(edges_hbm.at[pl.ds(chunk * (CHUNK * 6), CHUNK * 6)],
                         edg[b], sem_e[b])

    def drain_edges(b):
        pltpu.make_async_copy(edges_hbm.at[pl.ds(0, CHUNK * 6)],
                              edg[b], sem_e[b]).wait()

    def prep_and_fire(b):
        for g in range(CHUNK // 16):
            idx6 = (lax.iota(jnp.int32, 16) + g * 16) * 6
            r16 = plsc.load_gather(edg[b], [idx6])
            rel16 = plsc.load_gather(edg[b], [idx6 + 2])
            sub16 = plsc.load_gather(edg[b], [idx6 + 4])
            obj16 = jnp.minimum(plsc.load_gather(edg[b], [idx6 + 5]),
                                N_NODE - 1)
            sl = pl.ds(g * 16, 16)
            cols[b][0, sl] = sub16
            cols[b][1, sl] = rel16
            cols[b][2, sl] = r16
            obj[b][sl] = obj16
        pltpu.async_copy(subt_hbm.at[cols[b].at[0]],
                         sr[b].at[pl.ds(0, CHUNK)], sem_g[b])
        pltpu.async_copy(relt_hbm.at[cols[b].at[1]],
                         sr[b].at[pl.ds(CHUNK, CHUNK)], sem_g[b])
        pltpu.async_copy(tabC2_hbm.at[cols[b].at[2]], cb[b], sem_g[b])

    def drain_gathers(b):
        pltpu.make_async_copy(subt_hbm.at[pl.ds(0, 2 * CHUNK)],
                              sr[b], sem_g[b]).wait()
        pltpu.make_async_copy(tabC2_hbm.at[pl.ds(0, CHUNK)],
                              cb[b], sem_g[b]).wait()

    def compute_and_scatter(b):
        sr_v = sr[b]
        c_v = cb[b]
        fmt = plsc.PackFormat.INTERLEAVED
        bf16 = jnp.bfloat16

        def up(x):
            return plsc.unpack(plsc.bitcast(x, bf16), format=fmt,
                               preferred_element_type=jnp.float32)

        def edge_body(e, carry2):
            p = jnp.zeros((16,), jnp.float32)
            for k in range(4):
                asl = pl.ds(16 * k, 16)
                ae, ao = up(sr_v[e, asl])
                be, bo = up(sr_v[CHUNK + e, asl])
                te = ae + be + c_v[e, pl.ds(32 * k, 16)]
                to = ao + bo + c_v[e, pl.ds(32 * k + 16, 16)]
                p = p + jnp.maximum(te, 0.0) * wv[2 * k]
                p = p + jnp.maximum(to, 0.0) * wv[2 * k + 1]
            sval = jnp.sum(p)
            svec = jnp.full((16,), sval, jnp.float32) + sbv
            alpha = 1.0 / (1.0 + jnp.exp(-svec))
            for k in range(4):
                asl = pl.ds(64 + 16 * k, 16)
                he, ho = up(sr_v[e, asl])
                ge, go = up(sr_v[CHUNK + e, asl])
                m_v[e, pl.ds(32 * k, 16)] = alpha * (he + ge)
                m_v[e, pl.ds(32 * k + 16, 16)] = alpha * (ho + go)
            return carry2

        lax.fori_loop(0, CHUNK, edge_body, 0)
        pltpu.sync_copy(m_v, agg_sh.at[obj[b]], add=True)

    def valid(ii):
        return wid + ii * NW < NUM_CHUNKS

    fire_edges(0, 0)
    drain_edges(0)
    prep_and_fire(0)

    @pl.when(valid(1))
    def _():
        fire_edges(1, 1)

    def step(ii, b, nb):
        @pl.when(valid(ii + 2))
        def _():
            fire_edges(ii + 2, b)

        @pl.when(valid(ii + 1))
        def _():
            drain_edges(nb)
            prep_and_fire(nb)

        @pl.when(valid(ii))
        def _():
            drain_gathers(b)
            compute_and_scatter(b)

    def ring_body(i, carry):
        ii0 = 2 * i
        step(ii0, 0, 1)
        step(ii0 + 1, 1, 0)
        return carry

    lax.fori_loop(0, T_ITER // 2, ring_body, 0)
    plsc.subcore_barrier()

    @pl.when(s < 15)
    def _():
        pltpu.sync_copy(agg_sh.at[pl.ds(row0, 632)],
                        out_hbm.at[pl.ds(c * AGG + row0, 632)])

    @pl.when(s == 15)
    def _():
        pltpu.sync_copy(agg_sh.at[pl.ds(row0, 520)],
                        out_hbm.at[pl.ds(c * AGG + row0, 520)])


def kernel(q_sub, q_rel, hidden, edges, n_node, old_nodes_new_idx,
           rela_embed, Ws, Wr, Wqr, bqr, w_alpha_w, w_alpha_b, W_h):
    n_emb = rela_embed.shape[0]
    f32 = jnp.float32
    hidden = hidden.astype(f32)
    rela_p = jnp.concatenate(
        [rela_embed.astype(f32),
         jnp.zeros((R_PAD - n_emb, D), f32)], axis=0)

    perm = jnp.array(PERM, dtype=jnp.int32)
    wqr_p = jnp.take(Wqr.astype(f32), perm, axis=1)
    bqr_p = jnp.take(bqr.astype(f32), perm, axis=0)
    whp = jnp.take(W_h.astype(f32), perm, axis=0)

    subt = pl.pallas_call(
        _sub_body,
        grid=(25,),
        in_specs=[pl.BlockSpec((400, D), lambda i: (i, 0)),
                  pl.BlockSpec((D, D), lambda i: (0, 0))],
        out_specs=pl.BlockSpec((400, 2 * D), lambda i: (i, 0)),
        out_shape=jax.ShapeDtypeStruct((N_NODE, 2 * D), jnp.bfloat16),
    )(hidden, Ws.astype(f32))

    relt, tabC = pl.pallas_call(
        _rel_body,
        grid=(20,),
        in_specs=[pl.BlockSpec((512, D), lambda i: (i, 0)),
                  pl.BlockSpec((D, D), lambda i: (0, 0)),
                  pl.BlockSpec((D, D), lambda i: (0, 0)),
                  pl.BlockSpec((1, D), lambda i: (0, 0))],
        out_specs=[pl.BlockSpec((512, 2 * D), lambda i: (i, 0)),
                   pl.BlockSpec((512, D), lambda i: (i, 0))],
        out_shape=[jax.ShapeDtypeStruct((R_PAD, 2 * D), jnp.bfloat16),
                   jax.ShapeDtypeStruct((R_PAD, D), f32)],
    )(rela_p, Wr.astype(f32), wqr_p, bqr_p.reshape(1, D))

    subt3 = jax.lax.bitcast_convert_type(
        subt.reshape(N_NODE, D, 2), jnp.int32)
    relt3 = jax.lax.bitcast_convert_type(
        relt.reshape(R_PAD, D, 2), jnp.int32)

    mesh = plsc.VectorSubcoreMesh(core_axis_name="c", subcore_axis_name="s")
    sc_params = pltpu.CompilerParams(needs_layout_passes=False)

    q_rel_p = jnp.concatenate(
        [q_rel.astype(jnp.int32),
         jnp.zeros((B_PAD - q_rel.shape[0],), jnp.int32)])
    qgather = pl.kernel(
        _qgather_body,
        out_type=jax.ShapeDtypeStruct((B_PAD, D), f32),
        mesh=mesh,
        scratch_types=[
            pltpu.VMEM((QCHUNK,), jnp.int32),
            pltpu.VMEM((QCHUNK, D), f32),
            pltpu.SemaphoreType.DMA,
        ],
        compiler_params=sc_params,
    )
    tabC2 = qgather(tabC, q_rel_p)

    edges_flat = edges.astype(jnp.int32).reshape(-1)
    wvec = jnp.concatenate([jnp.take(w_alpha_w.astype(f32).reshape(-1),
                                     perm, axis=0),
                            jnp.broadcast_to(w_alpha_b.astype(f32), (1,))[0]
                            * jnp.ones((16,), f32)])

    sc_call = pl.kernel(
        _sc_body,
        out_type=jax.ShapeDtypeStruct((2 * AGG, D), f32),
        mesh=mesh,
        scratch_types=[
            pltpu.VMEM((CHUNK * 6,), jnp.int32),
            pltpu.VMEM((CHUNK * 6,), jnp.int32),
            pltpu.VMEM((3, CHUNK), jnp.int32),
            pltpu.VMEM((3, CHUNK), jnp.int32),
            pltpu.VMEM((CHUNK,), jnp.int32),
            pltpu.VMEM((CHUNK,), jnp.int32),
            pltpu.VMEM((2 * CHUNK, D), jnp.int32),
            pltpu.VMEM((2 * CHUNK, D), jnp.int32),
            pltpu.VMEM((CHUNK, D), f32),
            pltpu.VMEM((CHUNK, D), f32),
            pltpu.VMEM((CHUNK, D), f32),
            pltpu.VMEM((144,), f32),
            pltpu.VMEM_SHARED((AGG, D), f32),
            pltpu.SemaphoreType.DMA,
            pltpu.SemaphoreType.DMA,
            pltpu.SemaphoreType.DMA,
            pltpu.SemaphoreType.DMA,
        ],
        compiler_params=sc_params,
    )
    partial = sc_call(edges_flat, subt3, relt3, tabC2, wvec)

    part3 = partial.reshape(2, AGG, D)
    out = pl.pallas_call(
        _fin_body,
        grid=(25,),
        in_specs=[pl.BlockSpec((2, 400, D), lambda i: (0, i, 0)),
                  pl.BlockSpec((D, D), lambda i: (0, 0))],
        out_specs=pl.BlockSpec((400, D), lambda i: (i, 0)),
        out_shape=jax.ShapeDtypeStruct((N_NODE, D), f32),
    )(part3, whp.astype(f32))
    return out


# speedup vs baseline: 1.2171x; 1.2171x over previous
import jax
import jax.numpy as jnp
from jax import lax
from jax.experimental import pallas as pl
from jax.experimental.pallas import tpu as pltpu, tpu_sc as plsc

N_NODE = 10000
E = 320000
D = 128
R_PAD = 10240
B_PAD = 10240
CHUNK = 32
NUM_CHUNKS = E // CHUNK
NW = 32
T_ITER = 314
AGG = 10000
ROWS_PER_TILE = 632
C2_PER_TILE = B_PAD // 16


def _rnd16(v):
    u = jax.lax.bitcast_convert_type(v, jnp.uint32)
    return (u + jnp.uint32(0x7FFF) + ((u >> 16) & jnp.uint32(1))) >> 16


def _pack2(att, emb):
    return jax.lax.bitcast_convert_type(
        _rnd16(att) | (_rnd16(emb) << 16), jnp.int32)


def _sub_body(x_ref, w_ref, o_ref):
    x = x_ref[...]
    a = jnp.dot(x, w_ref[...], preferred_element_type=jnp.float32)
    o_ref[...] = _pack2(a, x)


def _rel_body(x_ref, wr_ref, wqr_ref, bqr_ref, or_ref, oc_ref):
    x = x_ref[...]
    a = jnp.dot(x, wr_ref[...], preferred_element_type=jnp.float32)
    or_ref[...] = _pack2(a, x)
    oc_ref[...] = (jnp.dot(x, wqr_ref[...], preferred_element_type=jnp.float32)
                   + bqr_ref[...])


def _fin_body(p_ref, w_ref, o_ref):
    x = p_ref[0] + p_ref[1]
    o_ref[...] = jnp.dot(x, w_ref[...], preferred_element_type=jnp.float32)


def _sc_body(edges_hbm, subt_hbm, relt_hbm, tabc_hbm, qrel_hbm, wvec_hbm,
             out_hbm, c2_hbm,
             edg0, edg1, cols0, cols1, obj0, obj1,
             sr0, sr1, c0, c1, m_v, w_v, agg_sh,
             sem_e0, sem_e1, sem_g0, sem_g1):
    c = lax.axis_index("c")
    s = lax.axis_index("s")
    wid = s * 2 + c

    edg = (edg0, edg1)
    cols = (cols0, cols1)
    obj = (obj0, obj1)
    sr = (sr0, sr1)
    cb = (c0, c1)
    sem_e = (sem_e0, sem_e1)
    sem_g = (sem_g0, sem_g1)

    pltpu.sync_copy(wvec_hbm, w_v)
    row0 = s * ROWS_PER_TILE
    c2base = c * B_PAD + s * C2_PER_TILE

    def c2_body(t, carry):
        off = s * C2_PER_TILE + t * CHUNK
        pltpu.sync_copy(qrel_hbm.at[pl.ds(off, CHUNK)], obj0)
        pltpu.async_copy(tabc_hbm.at[obj0], c0, sem_g0).wait()
        pltpu.sync_copy(c0, c2_hbm.at[pl.ds(c * B_PAD + off, CHUNK)])
        return carry

    lax.fori_loop(0, C2_PER_TILE // CHUNK, c2_body, 0)

    zf = jnp.zeros((16,), jnp.float32)

    def zero_body(e, cc):
        for k in range(8):
            m_v[e, pl.ds(16 * k, 16)] = zf
        return cc

    lax.fori_loop(0, CHUNK, zero_body, 0)

    @pl.when(s < 15)
    def _():
        for t in range(19):
            pltpu.sync_copy(m_v, agg_sh.at[pl.ds(row0 + 32 * t, 32)])
        pltpu.sync_copy(m_v.at[pl.ds(0, 24)],
                        agg_sh.at[pl.ds(row0 + 608, 24)])

    @pl.when(s == 15)
    def _():
        for t in range(16):
            pltpu.sync_copy(m_v, agg_sh.at[pl.ds(row0 + 32 * t, 32)])
        pltpu.sync_copy(m_v.at[pl.ds(0, 8)],
                        agg_sh.at[pl.ds(row0 + 512, 8)])

    plsc.subcore_barrier()

    wv = [w_v[pl.ds(16 * k, 16)] for k in range(8)]
    sbv = w_v[pl.ds(128, 16)]
    c2off = jnp.full((16,), 1, jnp.int32) * (c * B_PAD)

    def fire_edges(ii, b):
        chunk = wid + ii * NW
        pltpu.async_copy(edges_hbm.at[pl.ds(chunk * (CHUNK * 6), CHUNK * 6)],
                         edg[b], sem_e[b])

    def drain_edges(b):
        pltpu.make_async_copy(edges_hbm.at[pl.ds(0, CHUNK * 6)],
                              edg[b], sem_e[b]).wait()

    def prep_and_fire(b):
        for g in range(CHUNK // 16):
            idx6 = (lax.iota(jnp.int32, 16) + g * 16) * 6
            r16 = plsc.load_gather(edg[b], [idx6])
            rel16 = plsc.load_gather(edg[b], [idx6 + 2])
            sub16 = plsc.load_gather(edg[b], [idx6 + 4])
            obj16 = jnp.minimum(plsc.load_gather(edg[b], [idx6 + 5]),
                                N_NODE - 1)
            sl = pl.ds(g * 16, 16)
            cols[b][0, sl] = sub16
            cols[b][1, sl] = rel16
            cols[b][2, sl] = r16 + c2off
            obj[b][sl] = obj16
        pltpu.async_copy(subt_hbm.at[cols[b].at[0]],
                         sr[b].at[pl.ds(0, CHUNK)], sem_g[b])
        pltpu.async_copy(relt_hbm.at[cols[b].at[1]],
                         sr[b].at[pl.ds(CHUNK, CHUNK)], sem_g[b])
        pltpu.async_copy(c2_hbm.at[cols[b].at[2]], cb[b], sem_g[b])

    def drain_gathers(b):
        pltpu.make_async_copy(subt_hbm.at[pl.ds(0, 2 * CHUNK)],
                              sr[b], sem_g[b]).wait()
        pltpu.make_async_copy(tabc_hbm.at[pl.ds(0, CHUNK)],
                              cb[b], sem_g[b]).wait()

    def compute_and_scatter(b):
        sr_v = sr[b]
        c_v = cb[b]
        fmt = plsc.PackFormat.INTERLEAVED
        bf16 = jnp.bfloat16

        def up(x):
            return plsc.unpack(plsc.bitcast(x, bf16), format=fmt,
                               preferred_element_type=jnp.float32)

        def edge_body(e, carry2):
            p = jnp.zeros((16,), jnp.float32)
            hsum = []
            for k in range(8):
                ksl = pl.ds(16 * k, 16)
                aat, ahid = up(sr_v[e, ksl])
                bat, bhid = up(sr_v[CHUNK + e, ksl])
                t = aat + bat + c_v[e, ksl]
                p = p + jnp.maximum(t, 0.0) * wv[k]
                hsum.append(ahid + bhid)
            sval = jnp.sum(p)
            svec = jnp.full((16,), sval, jnp.float32) + sbv
            alpha = 1.0 / (1.0 + jnp.exp(-svec))
            for k in range(8):
                m_v[e, pl.ds(16 * k, 16)] = alpha * hsum[k]
            return carry2

        lax.fori_loop(0, CHUNK, edge_body, 0)
        pltpu.sync_copy(m_v, agg_sh.at[obj[b]], add=True)

    def valid(ii):
        return wid + ii * NW < NUM_CHUNKS

    fire_edges(0, 0)
    drain_edges(0)
    prep_and_fire(0)

    @pl.when(valid(1))
    def _():
        fire_edges(1, 1)

    def step(ii, b, nb):
        @pl.when(valid(ii + 2))
        def _():
            fire_edges(ii + 2, b)

        @pl.when(valid(ii + 1))
        def _():
            drain_edges(nb)
            prep_and_fire(nb)

        @pl.when(valid(ii))
        def _():
            drain_gathers(b)
            compute_and_scatter(b)

    def ring_body(i, carry):
        ii0 = 2 * i
        step(ii0, 0, 1)
        step(ii0 + 1, 1, 0)
        return carry

    lax.fori_loop(0, T_ITER // 2, ring_body, 0)
    plsc.subcore_barrier()

    @pl.when(s < 15)
    def _():
        pltpu.sync_copy(agg_sh.at[pl.ds(row0, 632)],
                        out_hbm.at[pl.ds(c * AGG + row0, 632)])

    @pl.when(s == 15)
    def _():
        pltpu.sync_copy(agg_sh.at[pl.ds(row0, 520)],
                        out_hbm.at[pl.ds(c * AGG + row0, 520)])


def kernel(q_sub, q_rel, hidden, edges, n_node, old_nodes_new_idx,
           rela_embed, Ws, Wr, Wqr, bqr, w_alpha_w, w_alpha_b, W_h):
    n_emb = rela_embed.shape[0]
    f32 = jnp.float32
    hidden = hidden.astype(f32)
    rela_p = jnp.concatenate(
        [rela_embed.astype(f32),
         jnp.zeros((R_PAD - n_emb, D), f32)], axis=0)

    subt = pl.pallas_call(
        _sub_body,
        grid=(25,),
        in_specs=[pl.BlockSpec((400, D), lambda i: (i, 0)),
                  pl.BlockSpec((D, D), lambda i: (0, 0))],
        out_specs=pl.BlockSpec((400, D), lambda i: (i, 0)),
        out_shape=jax.ShapeDtypeStruct((N_NODE, D), jnp.int32),
    )(hidden, Ws.astype(f32))

    relt, tabC = pl.pallas_call(
        _rel_body,
        grid=(20,),
        in_specs=[pl.BlockSpec((512, D), lambda i: (i, 0)),
                  pl.BlockSpec((D, D), lambda i: (0, 0)),
                  pl.BlockSpec((D, D), lambda i: (0, 0)),
                  pl.BlockSpec((1, D), lambda i: (0, 0))],
        out_specs=[pl.BlockSpec((512, D), lambda i: (i, 0)),
                   pl.BlockSpec((512, D), lambda i: (i, 0))],
        out_shape=[jax.ShapeDtypeStruct((R_PAD, D), jnp.int32),
                   jax.ShapeDtypeStruct((R_PAD, D), f32)],
    )(rela_p, Wr.astype(f32), Wqr.astype(f32), bqr.astype(f32).reshape(1, D))

    mesh = plsc.VectorSubcoreMesh(core_axis_name="c", subcore_axis_name="s")
    sc_params = pltpu.CompilerParams(needs_layout_passes=False)

    edges_flat = edges.astype(jnp.int32).reshape(-1)
    q_rel_p = jnp.concatenate(
        [q_rel.astype(jnp.int32),
         jnp.zeros((B_PAD - q_rel.shape[0],), jnp.int32)])
    wvec = jnp.concatenate([w_alpha_w.astype(f32).reshape(-1),
                            jnp.broadcast_to(w_alpha_b.astype(f32), (1,))[0]
                            * jnp.ones((16,), f32)])

    sc_call = pl.kernel(
        _sc_body,
        out_type=[jax.ShapeDtypeStruct((2 * AGG, D), f32),
                  jax.ShapeDtypeStruct((2 * B_PAD, D), f32)],
        mesh=mesh,
        scratch_types=[
            pltpu.VMEM((CHUNK * 6,), jnp.int32),
            pltpu.VMEM((CHUNK * 6,), jnp.int32),
            pltpu.VMEM((3, CHUNK), jnp.int32),
            pltpu.VMEM((3, CHUNK), jnp.int32),
            pltpu.VMEM((CHUNK,), jnp.int32),
            pltpu.VMEM((CHUNK,), jnp.int32),
            pltpu.VMEM((2 * CHUNK, D), jnp.int32),
            pltpu.VMEM((2 * CHUNK, D), jnp.int32),
            pltpu.VMEM((CHUNK, D), f32),
            pltpu.VMEM((CHUNK, D), f32),
            pltpu.VMEM((CHUNK, D), f32),
            pltpu.VMEM((144,), f32),
            pltpu.VMEM_SHARED((AGG, D), f32),
            pltpu.SemaphoreType.DMA,
            pltpu.SemaphoreType.DMA,
            pltpu.SemaphoreType.DMA,
            pltpu.SemaphoreType.DMA,
        ],
        compiler_params=sc_params,
    )
    partial, _ = sc_call(edges_flat, subt, relt, tabC, q_rel_p, wvec)

    part3 = partial.reshape(2, AGG, D)
    out = pl.pallas_call(
        _fin_body,
        grid=(25,),
        in_specs=[pl.BlockSpec((2, 400, D), lambda i: (0, i, 0)),
                  pl.BlockSpec((D, D), lambda i: (0, 0))],
        out_specs=pl.BlockSpec((400, D), lambda i: (i, 0)),
        out_shape=jax.ShapeDtypeStruct((N_NODE, D), f32),
    )(part3, W_h.astype(f32))
    return out


# speedup vs baseline: 1.3007x; 1.0687x over previous
import jax
import jax.numpy as jnp
from jax import lax
from jax.experimental import pallas as pl
from jax.experimental.pallas import tpu as pltpu, tpu_sc as plsc

N_NODE = 10000
E = 320000
D = 128
R_PAD = 10240
B_PAD = 10240
CHUNK = 32
NUM_CHUNKS = E // CHUNK
NW = 32
T_ITER = 314
AGG = 10000
ROWS_PER_TILE = 632
C2_PER_TILE = B_PAD // 16


def _rnd16(v):
    u = jax.lax.bitcast_convert_type(v, jnp.uint32)
    return (u + jnp.uint32(0x7FFF) + ((u >> 16) & jnp.uint32(1))) >> 16


def _pack2(att, emb):
    return jax.lax.bitcast_convert_type(
        _rnd16(att) | (_rnd16(emb) << 16), jnp.int32)


def _sub_body(x_ref, w_ref, o_ref):
    x = x_ref[...]
    a = jnp.dot(x, w_ref[...], preferred_element_type=jnp.float32)
    o_ref[...] = _pack2(a, x)


def _rel_body(x_ref, wr_ref, wqr_ref, bqr_ref, or_ref, oc_ref):
    x = x_ref[...]
    a = jnp.dot(x, wr_ref[...], preferred_element_type=jnp.float32)
    or_ref[...] = _pack2(a, x)
    oc_ref[...] = (jnp.dot(x, wqr_ref[...], preferred_element_type=jnp.float32)
                   + bqr_ref[...])


def _fin_body(p_ref, w_ref, o_ref):
    x = p_ref[0] + p_ref[1]
    o_ref[...] = jnp.dot(x, w_ref[...], preferred_element_type=jnp.float32)


def _sc_body(edges_hbm, subt_hbm, relt_hbm, tabc_hbm, qrel_hbm, wvec_hbm,
             out_hbm, c2_hbm,
             edg0, edg1, cols0, cols1, obj0, obj1, sobj0, sobj1,
             sr0, sr1, c0, c1, m0, m1, c2i, c2f, w_v, agg_sh,
             sem_e0, sem_e1, sem_g0, sem_g1, sem_s0, sem_s1):
    c = lax.axis_index("c")
    s = lax.axis_index("s")
    wid = s * 2 + c

    edg = (edg0, edg1)
    cols = (cols0, cols1)
    obj = (obj0, obj1)
    sobj = (sobj0, sobj1)
    sr = (sr0, sr1)
    cb = (c0, c1)
    msg = (m0, m1)
    sem_e = (sem_e0, sem_e1)
    sem_g = (sem_g0, sem_g1)
    sem_s = (sem_s0, sem_s1)

    pltpu.sync_copy(wvec_hbm, w_v)
    row0 = s * ROWS_PER_TILE

    def c2_body(t, carry):
        off = s * C2_PER_TILE + t * (2 * CHUNK)
        pltpu.sync_copy(qrel_hbm.at[pl.ds(off, 2 * CHUNK)], c2i)
        pltpu.async_copy(tabc_hbm.at[c2i], c2f, sem_g0).wait()
        pltpu.sync_copy(c2f, c2_hbm.at[pl.ds(c * B_PAD + off, 2 * CHUNK)])
        return carry

    lax.fori_loop(0, C2_PER_TILE // (2 * CHUNK), c2_body, 0)

    zf = jnp.zeros((16,), jnp.float32)

    def zero_body(e, cc):
        for k in range(8):
            m0[e, pl.ds(16 * k, 16)] = zf
        return cc

    lax.fori_loop(0, CHUNK, zero_body, 0)

    @pl.when(s < 15)
    def _():
        for t in range(19):
            pltpu.sync_copy(m0, agg_sh.at[pl.ds(row0 + 32 * t, 32)])
        pltpu.sync_copy(m0.at[pl.ds(0, 24)],
                        agg_sh.at[pl.ds(row0 + 608, 24)])

    @pl.when(s == 15)
    def _():
        for t in range(16):
            pltpu.sync_copy(m0, agg_sh.at[pl.ds(row0 + 32 * t, 32)])
        pltpu.sync_copy(m0.at[pl.ds(0, 8)],
                        agg_sh.at[pl.ds(row0 + 512, 8)])

    plsc.subcore_barrier()

    wv = [w_v[pl.ds(16 * k, 16)] for k in range(8)]
    sbv = w_v[pl.ds(128, 16)]
    c2off = jnp.full((16,), 1, jnp.int32) * (c * B_PAD)

    def fire_edges(ii, b):
        chunk = wid + ii * NW
        pltpu.async_copy(edges_hbm.at[pl.ds(chunk * (CHUNK * 6), CHUNK * 6)],
                         edg[b], sem_e[b])

    def drain_edges(b):
        pltpu.make_async_copy(edges_hbm.at[pl.ds(0, CHUNK * 6)],
                              edg[b], sem_e[b]).wait()

    def prep_and_fire(b):
        for g in range(CHUNK // 16):
            idx6 = (lax.iota(jnp.int32, 16) + g * 16) * 6
            r16 = plsc.load_gather(edg[b], [idx6])
            rel16 = plsc.load_gather(edg[b], [idx6 + 2])
            sub16 = plsc.load_gather(edg[b], [idx6 + 4])
            obj16 = jnp.minimum(plsc.load_gather(edg[b], [idx6 + 5]),
                                N_NODE - 1)
            sl = pl.ds(g * 16, 16)
            cols[b][0, sl] = sub16
            cols[b][1, sl] = rel16
            cols[b][2, sl] = r16 + c2off
            obj[b][sl] = obj16
        pltpu.async_copy(subt_hbm.at[cols[b].at[0]],
                         sr[b].at[pl.ds(0, CHUNK)], sem_g[b])
        pltpu.async_copy(relt_hbm.at[cols[b].at[1]],
                         sr[b].at[pl.ds(CHUNK, CHUNK)], sem_g[b])
        pltpu.async_copy(c2_hbm.at[cols[b].at[2]], cb[b], sem_g[b])

    def drain_gathers(b):
        pltpu.make_async_copy(subt_hbm.at[pl.ds(0, 2 * CHUNK)],
                              sr[b], sem_g[b]).wait()
        pltpu.make_async_copy(tabc_hbm.at[pl.ds(0, CHUNK)],
                              cb[b], sem_g[b]).wait()

    def compute_and_scatter(ii, b):
        sr_v = sr[b]
        c_v = cb[b]
        m_v = msg[b]
        fmt = plsc.PackFormat.INTERLEAVED
        bf16 = jnp.bfloat16

        @pl.when(ii >= 2)
        def _():
            pltpu.make_async_copy(m_v, agg_sh.at[sobj[b]],
                                  sem_s[b]).wait()

        def up(x):
            return plsc.unpack(plsc.bitcast(x, bf16), format=fmt,
                               preferred_element_type=jnp.float32)

        def edge_body(e, carry2):
            p = jnp.zeros((16,), jnp.float32)
            hsum = []
            for k in range(8):
                ksl = pl.ds(16 * k, 16)
                aat, ahid = up(sr_v[e, ksl])
                bat, bhid = up(sr_v[CHUNK + e, ksl])
                t = aat + bat + c_v[e, ksl]
                p = p + jnp.maximum(t, 0.0) * wv[k]
                hsum.append(ahid + bhid)
            sval = jnp.sum(p)
            svec = jnp.full((16,), sval, jnp.float32) + sbv
            alpha = 1.0 / (1.0 + jnp.exp(-svec))
            for k in range(8):
                m_v[e, pl.ds(16 * k, 16)] = alpha * hsum[k]
            return carry2

        lax.fori_loop(0, CHUNK, edge_body, 0)
        for g in range(CHUNK // 16):
            gsl = pl.ds(16 * g, 16)
            sobj[b][gsl] = obj[b][gsl]
        pltpu.async_copy(m_v, agg_sh.at[sobj[b]], sem_s[b], add=True)

    def valid(ii):
        return wid + ii * NW < NUM_CHUNKS

    fire_edges(0, 0)
    drain_edges(0)
    prep_and_fire(0)

    @pl.when(valid(1))
    def _():
        fire_edges(1, 1)

    def step(ii, b, nb):
        @pl.when(valid(ii + 2))
        def _():
            fire_edges(ii + 2, b)

        @pl.when(valid(ii + 1))
        def _():
            drain_edges(nb)
            prep_and_fire(nb)

        @pl.when(valid(ii))
        def _():
            drain_gathers(b)
            compute_and_scatter(ii, b)

    def ring_body(i, carry):
        ii0 = 2 * i
        step(ii0, 0, 1)
        step(ii0 + 1, 1, 0)
        return carry

    lax.fori_loop(0, T_ITER // 2, ring_body, 0)

    for b in range(2):
        @pl.when(valid(b))
        def _():
            pltpu.make_async_copy(msg[b], agg_sh.at[sobj[b]],
                                  sem_s[b]).wait()

    plsc.subcore_barrier()

    @pl.when(s < 15)
    def _():
        pltpu.sync_copy(agg_sh.at[pl.ds(row0, 632)],
                        out_hbm.at[pl.ds(c * AGG + row0, 632)])

    @pl.when(s == 15)
    def _():
        pltpu.sync_copy(agg_sh.at[pl.ds(row0, 520)],
                        out_hbm.at[pl.ds(c * AGG + row0, 520)])


def kernel(q_sub, q_rel, hidden, edges, n_node, old_nodes_new_idx,
           rela_embed, Ws, Wr, Wqr, bqr, w_alpha_w, w_alpha_b, W_h):
    n_emb = rela_embed.shape[0]
    f32 = jnp.float32
    hidden = hidden.astype(f32)
    rela_p = jnp.concatenate(
        [rela_embed.astype(f32),
         jnp.zeros((R_PAD - n_emb, D), f32)], axis=0)

    subt = pl.pallas_call(
        _sub_body,
        grid=(25,),
        in_specs=[pl.BlockSpec((400, D), lambda i: (i, 0)),
                  pl.BlockSpec((D, D), lambda i: (0, 0))],
        out_specs=pl.BlockSpec((400, D), lambda i: (i, 0)),
        out_shape=jax.ShapeDtypeStruct((N_NODE, D), jnp.int32),
    )(hidden, Ws.astype(f32))

    relt, tabC = pl.pallas_call(
        _rel_body,
        grid=(20,),
        in_specs=[pl.BlockSpec((512, D), lambda i: (i, 0)),
                  pl.BlockSpec((D, D), lambda i: (0, 0)),
                  pl.BlockSpec((D, D), lambda i: (0, 0)),
                  pl.BlockSpec((1, D), lambda i: (0, 0))],
        out_specs=[pl.BlockSpec((512, D), lambda i: (i, 0)),
                   pl.BlockSpec((512, D), lambda i: (i, 0))],
        out_shape=[jax.ShapeDtypeStruct((R_PAD, D), jnp.int32),
                   jax.ShapeDtypeStruct((R_PAD, D), f32)],
    )(rela_p, Wr.astype(f32), Wqr.astype(f32), bqr.astype(f32).reshape(1, D))

    mesh = plsc.VectorSubcoreMesh(core_axis_name="c", subcore_axis_name="s")
    sc_params = pltpu.CompilerParams(needs_layout_passes=False)

    edges_flat = edges.astype(jnp.int32).reshape(-1)
    q_rel_p = jnp.concatenate(
        [q_rel.astype(jnp.int32),
         jnp.zeros((B_PAD - q_rel.shape[0],), jnp.int32)])
    wvec = jnp.concatenate([w_alpha_w.astype(f32).reshape(-1),
                            jnp.broadcast_to(w_alpha_b.astype(f32), (1,))[0]
                            * jnp.ones((16,), f32)])

    sc_call = pl.kernel(
        _sc_body,
        out_type=[jax.ShapeDtypeStruct((2 * AGG, D), f32),
                  jax.ShapeDtypeStruct((2 * B_PAD, D), f32)],
        mesh=mesh,
        scratch_types=[
            pltpu.VMEM((CHUNK * 6,), jnp.int32),
            pltpu.VMEM((CHUNK * 6,), jnp.int32),
            pltpu.VMEM((3, CHUNK), jnp.int32),
            pltpu.VMEM((3, CHUNK), jnp.int32),
            pltpu.VMEM((CHUNK,), jnp.int32),
            pltpu.VMEM((CHUNK,), jnp.int32),
            pltpu.VMEM((CHUNK,), jnp.int32),
            pltpu.VMEM((CHUNK,), jnp.int32),
            pltpu.VMEM((2 * CHUNK, D), jnp.int32),
            pltpu.VMEM((2 * CHUNK, D), jnp.int32),
            pltpu.VMEM((CHUNK, D), f32),
            pltpu.VMEM((CHUNK, D), f32),
            pltpu.VMEM((CHUNK, D), f32),
            pltpu.VMEM((CHUNK, D), f32),
            pltpu.VMEM((2 * CHUNK,), jnp.int32),
            pltpu.VMEM((2 * CHUNK, D), f32),
            pltpu.VMEM((144,), f32),
            pltpu.VMEM_SHARED((AGG, D), f32),
            pltpu.SemaphoreType.DMA,
            pltpu.SemaphoreType.DMA,
            pltpu.SemaphoreType.DMA,
            pltpu.SemaphoreType.DMA,
            pltpu.SemaphoreType.DMA,
            pltpu.SemaphoreType.DMA,
        ],
        compiler_params=sc_params,
    )
    partial, _ = sc_call(edges_flat, subt, relt, tabC, q_rel_p, wvec)

    part3 = partial.reshape(2, AGG, D)
    out = pl.pallas_call(
        _fin_body,
        grid=(25,),
        in_specs=[pl.BlockSpec((2, 400, D), lambda i: (0, i, 0)),
                  pl.BlockSpec((D, D), lambda i: (0, 0))],
        out_specs=pl.BlockSpec((400, D), lambda i: (i, 0)),
        out_shape=jax.ShapeDtypeStruct((N_NODE, D), f32),
    )(part3, W_h.astype(f32))
    return out


# speedup vs baseline: 1.3117x; 1.0084x over previous
import jax
import jax.numpy as jnp
from jax import lax
from jax.experimental import pallas as pl
from jax.experimental.pallas import tpu as pltpu, tpu_sc as plsc

N_NODE = 10000
E = 320000
D = 128
R_PAD = 10240
B_PAD = 10240
CHUNK = 32
NUM_CHUNKS = E // CHUNK
NW = 32
T_ITER = 314
AGG = 10000
ROWS_PER_TILE = 632
C2_PER_TILE = B_PAD // 16


def _rnd16(v):
    u = jax.lax.bitcast_convert_type(v, jnp.uint32)
    return (u + jnp.uint32(0x7FFF) + ((u >> 16) & jnp.uint32(1))) >> 16


def _pack2(att, emb):
    return jax.lax.bitcast_convert_type(
        _rnd16(att) | (_rnd16(emb) << 16), jnp.int32)


def _tab_body(hid_ref, rela_ref, ws_ref, wr_ref, wqr_ref, bqr_ref,
              osub_ref, orel_ref, oc_ref):
    i = pl.program_id(0)

    @pl.when(i < 25)
    def _():
        x = hid_ref[...]
        a = jnp.dot(x, ws_ref[...], preferred_element_type=jnp.float32)
        osub_ref[...] = _pack2(a, x)

    @pl.when(i >= 25)
    def _():
        x = rela_ref[...]
        a = jnp.dot(x, wr_ref[...], preferred_element_type=jnp.float32)
        orel_ref[...] = _pack2(a, x)
        oc_ref[...] = (jnp.dot(x, wqr_ref[...],
                               preferred_element_type=jnp.float32)
                       + bqr_ref[...])


def _fin_body(p_ref, w_ref, o_ref):
    x = p_ref[0] + p_ref[1]
    o_ref[...] = jnp.dot(x, w_ref[...], preferred_element_type=jnp.float32)


def _sc_body(edges_hbm, subt_hbm, relt_hbm, tabc_hbm, qrel_hbm, wvec_hbm,
             out_hbm, c2_hbm,
             edg0, edg1, cols0, cols1, obj0, obj1, sobj0, sobj1,
             sr0, sr1, c0, c1, m0, m1, c2i, c2f, w_v, agg_sh,
             sem_e0, sem_e1, sem_g0, sem_g1, sem_s0, sem_s1):
    c = lax.axis_index("c")
    s = lax.axis_index("s")
    wid = s * 2 + c

    edg = (edg0, edg1)
    cols = (cols0, cols1)
    obj = (obj0, obj1)
    sobj = (sobj0, sobj1)
    sr = (sr0, sr1)
    cb = (c0, c1)
    msg = (m0, m1)
    sem_e = (sem_e0, sem_e1)
    sem_g = (sem_g0, sem_g1)
    sem_s = (sem_s0, sem_s1)

    pltpu.sync_copy(wvec_hbm, w_v)
    row0 = s * ROWS_PER_TILE

    def c2_body(t, carry):
        off = s * C2_PER_TILE + t * (2 * CHUNK)
        pltpu.sync_copy(qrel_hbm.at[pl.ds(off, 2 * CHUNK)], c2i)
        pltpu.async_copy(tabc_hbm.at[c2i], c2f, sem_g0).wait()
        pltpu.sync_copy(c2f, c2_hbm.at[pl.ds(c * B_PAD + off, 2 * CHUNK)])
        return carry

    lax.fori_loop(0, C2_PER_TILE // (2 * CHUNK), c2_body, 0)

    zf = jnp.zeros((16,), jnp.float32)

    def zero_body(e, cc):
        for k in range(8):
            m0[e, pl.ds(16 * k, 16)] = zf
        return cc

    lax.fori_loop(0, CHUNK, zero_body, 0)

    @pl.when(s < 15)
    def _():
        for t in range(19):
            pltpu.sync_copy(m0, agg_sh.at[pl.ds(row0 + 32 * t, 32)])
        pltpu.sync_copy(m0.at[pl.ds(0, 24)],
                        agg_sh.at[pl.ds(row0 + 608, 24)])

    @pl.when(s == 15)
    def _():
        for t in range(16):
            pltpu.sync_copy(m0, agg_sh.at[pl.ds(row0 + 32 * t, 32)])
        pltpu.sync_copy(m0.at[pl.ds(0, 8)],
                        agg_sh.at[pl.ds(row0 + 512, 8)])

    plsc.subcore_barrier()

    wv = [w_v[pl.ds(16 * k, 16)] for k in range(8)]
    sbv = w_v[pl.ds(128, 16)]
    c2off = jnp.full((16,), 1, jnp.int32) * (c * B_PAD)

    def fire_edges(ii, b):
        chunk = wid + ii * NW
        pltpu.async_copy(edges_hbm.at[pl.ds(chunk * (CHUNK * 6), CHUNK * 6)],
                         edg[b], sem_e[b])

    def drain_edges(b):
        pltpu.make_async_copy(edges_hbm.at[pl.ds(0, CHUNK * 6)],
                              edg[b], sem_e[b]).wait()

    def prep_and_fire(b):
        for g in range(CHUNK // 16):
            idx6 = (lax.iota(jnp.int32, 16) + g * 16) * 6
            r16 = plsc.load_gather(edg[b], [idx6])
            rel16 = plsc.load_gather(edg[b], [idx6 + 2])
            sub16 = plsc.load_gather(edg[b], [idx6 + 4])
            obj16 = jnp.minimum(plsc.load_gather(edg[b], [idx6 + 5]),
                                N_NODE - 1)
            sl = pl.ds(g * 16, 16)
            cols[b][0, sl] = sub16
            cols[b][1, sl] = rel16
            cols[b][2, sl] = r16 + c2off
            obj[b][sl] = obj16
        pltpu.async_copy(subt_hbm.at[cols[b].at[0]],
                         sr[b].at[pl.ds(0, CHUNK)], sem_g[b])
        pltpu.async_copy(relt_hbm.at[cols[b].at[1]],
                         sr[b].at[pl.ds(CHUNK, CHUNK)], sem_g[b])
        pltpu.async_copy(c2_hbm.at[cols[b].at[2]], cb[b], sem_g[b])

    def drain_gathers(b):
        pltpu.make_async_copy(subt_hbm.at[pl.ds(0, 2 * CHUNK)],
                              sr[b], sem_g[b]).wait()
        pltpu.make_async_copy(tabc_hbm.at[pl.ds(0, CHUNK)],
                              cb[b], sem_g[b]).wait()

    def compute_and_scatter(ii, b):
        sr_v = sr[b]
        c_v = cb[b]
        m_v = msg[b]
        fmt = plsc.PackFormat.INTERLEAVED
        bf16 = jnp.bfloat16

        @pl.when(ii >= 2)
        def _():
            pltpu.make_async_copy(m_v, agg_sh.at[sobj[b]],
                                  sem_s[b]).wait()

        def up(x):
            return plsc.unpack(plsc.bitcast(x, bf16), format=fmt,
                               preferred_element_type=jnp.float32)

        def edge_body(e, carry2):
            p = jnp.zeros((16,), jnp.float32)
            hsum = []
            for k in range(8):
                ksl = pl.ds(16 * k, 16)
                aat, ahid = up(sr_v[e, ksl])
                bat, bhid = up(sr_v[CHUNK + e, ksl])
                t = aat + bat + c_v[e, ksl]
                p = p + jnp.maximum(t, 0.0) * wv[k]
                hsum.append(ahid + bhid)
            sval = jnp.sum(p)
            svec = jnp.full((16,), sval, jnp.float32) + sbv
            alpha = 1.0 / (1.0 + jnp.exp(-svec))
            for k in range(8):
                m_v[e, pl.ds(16 * k, 16)] = alpha * hsum[k]
            return carry2

        lax.fori_loop(0, CHUNK, edge_body, 0)
        for g in range(CHUNK // 16):
            gsl = pl.ds(16 * g, 16)
            sobj[b][gsl] = obj[b][gsl]
        pltpu.async_copy(m_v, agg_sh.at[sobj[b]], sem_s[b], add=True)

    def valid(ii):
        return wid + ii * NW < NUM_CHUNKS

    fire_edges(0, 0)
    drain_edges(0)
    prep_and_fire(0)

    @pl.when(valid(1))
    def _():
        fire_edges(1, 1)

    def step(ii, b, nb):
        @pl.when(valid(ii + 2))
        def _():
            fire_edges(ii + 2, b)

        @pl.when(valid(ii + 1))
        def _():
            drain_edges(nb)
            prep_and_fire(nb)

        @pl.when(valid(ii))
        def _():
            drain_gathers(b)
            compute_and_scatter(ii, b)

    def ring_body(i, carry):
        ii0 = 2 * i
        step(ii0, 0, 1)
        step(ii0 + 1, 1, 0)
        return carry

    lax.fori_loop(0, T_ITER // 2, ring_body, 0)

    for b in range(2):
        @pl.when(valid(b))
        def _():
            pltpu.make_async_copy(msg[b], agg_sh.at[sobj[b]],
                                  sem_s[b]).wait()

    plsc.subcore_barrier()

    @pl.when(s < 15)
    def _():
        pltpu.sync_copy(agg_sh.at[pl.ds(row0, 632)],
                        out_hbm.at[pl.ds(c * AGG + row0, 632)])

    @pl.when(s == 15)
    def _():
        pltpu.sync_copy(agg_sh.at[pl.ds(row0, 520)],
                        out_hbm.at[pl.ds(c * AGG + row0, 520)])


def kernel(q_sub, q_rel, hidden, edges, n_node, old_nodes_new_idx,
           rela_embed, Ws, Wr, Wqr, bqr, w_alpha_w, w_alpha_b, W_h):
    n_emb = rela_embed.shape[0]
    f32 = jnp.float32
    hidden = hidden.astype(f32)
    rela = rela_embed.astype(f32)

    subt, relt, tabC = pl.pallas_call(
        _tab_body,
        grid=(45,),
        in_specs=[
            pl.BlockSpec((400, D), lambda i: (jnp.minimum(i, 24), 0)),
            pl.BlockSpec((512, D),
                         lambda i: (jnp.clip(i - 25, 0, 19), 0)),
            pl.BlockSpec((D, D), lambda i: (0, 0)),
            pl.BlockSpec((D, D), lambda i: (0, 0)),
            pl.BlockSpec((D, D), lambda i: (0, 0)),
            pl.BlockSpec((1, D), lambda i: (0, 0)),
        ],
        out_specs=[
            pl.BlockSpec((400, D), lambda i: (jnp.minimum(i, 24), 0)),
            pl.BlockSpec((512, D), lambda i: (jnp.clip(i - 25, 0, 19), 0)),
            pl.BlockSpec((512, D), lambda i: (jnp.clip(i - 25, 0, 19), 0)),
        ],
        out_shape=[jax.ShapeDtypeStruct((N_NODE, D), jnp.int32),
                   jax.ShapeDtypeStruct((R_PAD, D), jnp.int32),
                   jax.ShapeDtypeStruct((R_PAD, D), f32)],
    )(hidden, rela, Ws.astype(f32), Wr.astype(f32), Wqr.astype(f32),
      bqr.astype(f32).reshape(1, D))

    mesh = plsc.VectorSubcoreMesh(core_axis_name="c", subcore_axis_name="s")
    sc_params = pltpu.CompilerParams(needs_layout_passes=False)

    edges_flat = edges.astype(jnp.int32).reshape(-1)
    q_rel_p = jnp.concatenate(
        [q_rel.astype(jnp.int32),
         jnp.zeros((B_PAD - q_rel.shape[0],), jnp.int32)])
    wvec = jnp.concatenate([w_alpha_w.astype(f32).reshape(-1),
                            jnp.broadcast_to(w_alpha_b.astype(f32), (1,))[0]
                            * jnp.ones((16,), f32)])

    sc_call = pl.kernel(
        _sc_body,
        out_type=[jax.ShapeDtypeStruct((2 * AGG, D), f32),
                  jax.ShapeDtypeStruct((2 * B_PAD, D), f32)],
        mesh=mesh,
        scratch_types=[
            pltpu.VMEM((CHUNK * 6,), jnp.int32),
            pltpu.VMEM((CHUNK * 6,), jnp.int32),
            pltpu.VMEM((3, CHUNK), jnp.int32),
            pltpu.VMEM((3, CHUNK), jnp.int32),
            pltpu.VMEM((CHUNK,), jnp.int32),
            pltpu.VMEM((CHUNK,), jnp.int32),
            pltpu.VMEM((CHUNK,), jnp.int32),
            pltpu.VMEM((CHUNK,), jnp.int32),
            pltpu.VMEM((2 * CHUNK, D), jnp.int32),
            pltpu.VMEM((2 * CHUNK, D), jnp.int32),
            pltpu.VMEM((CHUNK, D), f32),
            pltpu.VMEM((CHUNK, D), f32),
            pltpu.VMEM((CHUNK, D), f32),
            pltpu.VMEM((CHUNK, D), f32),
            pltpu.VMEM((2 * CHUNK,), jnp.int32),
            pltpu.VMEM((2 * CHUNK, D), f32),
            pltpu.VMEM((144,), f32),
            pltpu.VMEM_SHARED((AGG, D), f32),
            pltpu.SemaphoreType.DMA,
            pltpu.SemaphoreType.DMA,
            pltpu.SemaphoreType.DMA,
            pltpu.SemaphoreType.DMA,
            pltpu.SemaphoreType.DMA,
            pltpu.SemaphoreType.DMA,
        ],
        compiler_params=sc_params,
    )
    partial, _ = sc_call(edges_flat, subt, relt, tabC, q_rel_p, wvec)

    part3 = partial.reshape(2, AGG, D)
    out = pl.pallas_call(
        _fin_body,
        grid=(25,),
        in_specs=[pl.BlockSpec((2, 400, D), lambda i: (0, i, 0)),
                  pl.BlockSpec((D, D), lambda i: (0, 0))],
        out_specs=pl.BlockSpec((400, D), lambda i: (i, 0)),
        out_shape=jax.ShapeDtypeStruct((N_NODE, D), f32),
    )(part3, W_h.astype(f32))
    return out
